# baseline (speedup 1.0000x reference)
import functools

import jax
import jax.numpy as jnp
from jax import lax
from jax.experimental import pallas as pl
from jax.experimental.pallas import tpu as pltpu
from jax.experimental.pallas import tpu_sc as plsc

EMB = 64
FIELDS = 26
BATCH = 16384
NBLK = BATCH // 128
NC = 2
NS = 16
NW = NC * NS
CPW = NBLK // NW
UPW = CPW * FIELDS


@jax.jit
def _sc_embed(wpair, idx_t):
    mesh = plsc.VectorSubcoreMesh(core_axis_name="c", subcore_axis_name="s")

    @functools.partial(
        pl.kernel,
        out_type=jax.ShapeDtypeStruct((FIELDS * EMB, BATCH), jnp.float32),
        mesh=mesh,
        scratch_types=(
            pltpu.VMEM((UPW, 128), jnp.int32),
            pltpu.VMEM((128,), jnp.int32),
            pltpu.VMEM((128,), jnp.int32),
            pltpu.VMEM((128,), jnp.int32),
            pltpu.VMEM((128,), jnp.int32),
            pltpu.VMEM((128, 128), jnp.float32),
            pltpu.VMEM((128, 128), jnp.float32),
            pltpu.VMEM((EMB, 128), jnp.float32),
            pltpu.VMEM((EMB, 128), jnp.float32),
            pltpu.SemaphoreType.DMA,
            pltpu.SemaphoreType.DMA,
            pltpu.SemaphoreType.DMA,
            pltpu.SemaphoreType.DMA,
            pltpu.SemaphoreType.DMA,
        ),
        compiler_params=pltpu.CompilerParams(
            use_tc_tiling_on_sc=True, needs_layout_passes=False),
    )
    def k(wp_hbm, idx_hbm, out_hbm, idxall, pairb0, pairb1, offb0, offb1,
          gbuf0, gbuf1, tbuf0, tbuf1, i_sem, g_s0, g_s1, o_s0, o_s1):
        pairb = (pairb0, pairb1)
        offb = (offb0, offb1)
        gbuf = (gbuf0, gbuf1)
        tbuf = (tbuf0, tbuf1)
        g_s = (g_s0, g_s1)
        o_s = (o_s0, o_s1)
        wid = lax.axis_index("s") * NC + lax.axis_index("c")

        for ci in range(CPW):
            col = (wid + NW * ci) * 128
            pltpu.async_copy(
                idx_hbm.at[pl.ds(0, FIELDS), pl.ds(col, 128)],
                idxall.at[pl.ds(ci * FIELDS, FIELDS)], i_sem)
        for ci in range(CPW):
            col = (wid + NW * ci) * 128
            pltpu.make_async_copy(
                idx_hbm.at[pl.ds(0, FIELDS), pl.ds(col, 128)],
                idxall.at[pl.ds(ci * FIELDS, FIELDS)], i_sem).wait()

        def unit_fc(t):
            t = jnp.asarray(t, jnp.int32)
            ci = ((t >= FIELDS).astype(jnp.int32)
                  + (t >= 2 * FIELDS).astype(jnp.int32)
                  + (t >= 3 * FIELDS).astype(jnp.int32))
            f = t - FIELDS * ci
            return ci, f

        def out_slice(t):
            ci, f = unit_fc(t)
            col = (wid + NW * ci) * 128
            return out_hbm.at[pl.ds(f * EMB, EMB), pl.ds(col, 128)]

        def process(tp, ob):
            pltpu.make_async_copy(
                wp_hbm.at[pairb[ob]], gbuf[ob], g_s[ob]).wait()

            @pl.when(jnp.asarray(tp >= 2))
            def _():
                pltpu.make_async_copy(
                    tbuf[ob], out_slice(tp - 2), o_s[ob]).wait()

            colv = [offb[ob][pl.ds(16 * g, 16)] for g in range(8)]
            base = lax.iota(jnp.int32, 16)

            @pl.loop(0, EMB, unroll=4)
            def _(d):
                for g in range(8):
                    row = base + 16 * g
                    col = colv[g] + d
                    vals = plsc.load_gather(gbuf[ob], [row, col])
                    tbuf[ob][d, pl.ds(16 * g, 16)] = vals

            pltpu.async_copy(tbuf[ob], out_slice(tp), o_s[ob])

        def stage(t, b):
            for g in range(8):
                iv = idxall[t, pl.ds(16 * g, 16)]
                pairb[b][pl.ds(16 * g, 16)] = iv >> 1
                offb[b][pl.ds(16 * g, 16)] = (iv & 1) << 6
            pltpu.async_copy(wp_hbm.at[pairb[b]], gbuf[b], g_s[b])

            @pl.when(jnp.asarray(t > 0))
            def _():
                process(t - 1, 1 - b)

        @pl.loop(0, UPW, step=2)
        def _(tt):
            stage(tt, 0)
            stage(tt + 1, 1)

        process(UPW - 1, 1)
        pltpu.make_async_copy(tbuf[0], out_slice(UPW - 2), o_s[0]).wait()
        pltpu.make_async_copy(tbuf[1], out_slice(UPW - 1), o_s[1]).wait()

    return k(wpair, idx_t)


def kernel(indices, weight):
    wpair = weight.reshape(500000, 128)
    idx_t = indices.astype(jnp.int32).T
    out2 = _sc_embed(wpair, idx_t)
    out3 = out2.reshape(FIELDS, EMB, BATCH)
    return out3.transpose(2, 0, 1)


# speedup vs baseline: 1.7201x; 1.7201x over previous
import functools

import jax
import jax.numpy as jnp
from jax import lax
from jax.experimental import pallas as pl
from jax.experimental.pallas import tpu as pltpu
from jax.experimental.pallas import tpu_sc as plsc

EMB = 64
FIELDS = 26
BATCH = 16384
NBLK = BATCH // 128
NC = 2
NS = 16
NW = NC * NS
CPW = NBLK // NW
UPW = CPW * FIELDS


@jax.jit
def _sc_embed(wpair, idx_t):
    mesh = plsc.VectorSubcoreMesh(core_axis_name="c", subcore_axis_name="s")

    @functools.partial(
        pl.kernel,
        out_type=jax.ShapeDtypeStruct((FIELDS * EMB, BATCH), jnp.float32),
        mesh=mesh,
        scratch_types=(
            pltpu.VMEM((UPW, 128), jnp.int32),
            pltpu.VMEM((128,), jnp.int32),
            pltpu.VMEM((128,), jnp.int32),
            pltpu.VMEM((128,), jnp.int32),
            pltpu.VMEM((128,), jnp.int32),
            pltpu.VMEM((128, 128), jnp.float32),
            pltpu.VMEM((128, 128), jnp.float32),
            pltpu.VMEM((EMB, 128), jnp.float32),
            pltpu.VMEM((EMB, 128), jnp.float32),
            pltpu.SemaphoreType.DMA,
            pltpu.SemaphoreType.DMA,
            pltpu.SemaphoreType.DMA,
            pltpu.SemaphoreType.DMA,
            pltpu.SemaphoreType.DMA,
        ),
        compiler_params=pltpu.CompilerParams(
            use_tc_tiling_on_sc=True, needs_layout_passes=False),
    )
    def k(wp_hbm, idx_hbm, out_hbm, idxall, pairb0, pairb1, offb0, offb1,
          gbuf0, gbuf1, tbuf0, tbuf1, i_sem, g_s0, g_s1, o_s0, o_s1):
        pairb = (pairb0, pairb1)
        offb = (offb0, offb1)
        gbuf = (gbuf0, gbuf1)
        tbuf = (tbuf0, tbuf1)
        g_s = (g_s0, g_s1)
        o_s = (o_s0, o_s1)
        wid = lax.axis_index("s") * NC + lax.axis_index("c")

        for ci in range(CPW):
            col = (wid + NW * ci) * 128
            pltpu.async_copy(
                idx_hbm.at[pl.ds(0, FIELDS), pl.ds(col, 128)],
                idxall.at[pl.ds(ci * FIELDS, FIELDS)], i_sem)
        for ci in range(CPW):
            col = (wid + NW * ci) * 128
            pltpu.make_async_copy(
                idx_hbm.at[pl.ds(0, FIELDS), pl.ds(col, 128)],
                idxall.at[pl.ds(ci * FIELDS, FIELDS)], i_sem).wait()

        def unit_fc(t):
            t = jnp.asarray(t, jnp.int32)
            ci = ((t >= FIELDS).astype(jnp.int32)
                  + (t >= 2 * FIELDS).astype(jnp.int32)
                  + (t >= 3 * FIELDS).astype(jnp.int32))
            f = t - FIELDS * ci
            return ci, f

        def out_slice(t):
            ci, f = unit_fc(t)
            col = (wid + NW * ci) * 128
            return out_hbm.at[pl.ds(f * EMB, EMB), pl.ds(col, 128)]

        def process(tp, ob):
            pltpu.make_async_copy(
                wp_hbm.at[pairb[ob]], gbuf[ob], g_s[ob]).wait()

            @pl.when(jnp.asarray(tp >= 2))
            def _():
                pltpu.make_async_copy(
                    tbuf[ob], out_slice(tp - 2), o_s[ob]).wait()

            base = lax.iota(jnp.int32, 16)
            rows = [base + 16 * g for g in range(8)]
            colv = [offb[ob][pl.ds(16 * g, 16)] for g in range(8)]

            @functools.partial(plsc.parallel_loop, 0, EMB, unroll=8)
            def _(d):
                vals = [plsc.load_gather(gbuf[ob], [rows[g], colv[g] + d])
                        for g in range(8)]
                for g in range(8):
                    tbuf[ob][d, pl.ds(16 * g, 16)] = vals[g]

            pltpu.async_copy(tbuf[ob], out_slice(tp), o_s[ob])

        def stage(t, b):
            for g in range(8):
                iv = idxall[t, pl.ds(16 * g, 16)]
                pairb[b][pl.ds(16 * g, 16)] = iv >> 1
                offb[b][pl.ds(16 * g, 16)] = (iv & 1) << 6
            pltpu.async_copy(wp_hbm.at[pairb[b]], gbuf[b], g_s[b])

            @pl.when(jnp.asarray(t > 0))
            def _():
                process(t - 1, 1 - b)

        @pl.loop(0, UPW, step=2)
        def _(tt):
            stage(tt, 0)
            stage(tt + 1, 1)

        process(UPW - 1, 1)
        pltpu.make_async_copy(tbuf[0], out_slice(UPW - 2), o_s[0]).wait()
        pltpu.make_async_copy(tbuf[1], out_slice(UPW - 1), o_s[1]).wait()

    return k(wpair, idx_t)


def kernel(indices, weight):
    wpair = weight.reshape(500000, 128)
    idx_t = indices.astype(jnp.int32).T
    out2 = _sc_embed(wpair, idx_t)
    out3 = out2.reshape(FIELDS, EMB, BATCH)
    return out3.transpose(2, 0, 1)
